# baseline (speedup 1.0000x reference)
import functools

import jax
import jax.numpy as jnp
from jax.experimental import pallas as pl
from jax.experimental.pallas import tpu as pltpu

B = 8
N = 8192
A = 6
G = 256
M = 32


def _fps_body(xyz_ref, idx_ref, cattr_ref):
    X = xyz_ref[0]
    Y = xyz_ref[1]
    Z = xyz_ref[2]
    lane = jax.lax.broadcasted_iota(jnp.int32, (B, N), 1)

    def step(i, carry):
        distance, far, idx_acc, cattr_acc = carry
        oh = lane == far
        idx_acc = jnp.concatenate([idx_acc[:, 1:], far], axis=1)
        cs = [jnp.sum(jnp.where(oh, xyz_ref[a], 0.0), axis=1, keepdims=True)
              for a in range(A)]
        cattr_acc = jnp.concatenate([cattr_acc[:, :, 1:], jnp.stack(cs)],
                                    axis=2)
        dx = X - cs[0]
        dy = Y - cs[1]
        dz = Z - cs[2]
        dist = (dx * dx + dy * dy) + dz * dz
        distance = jnp.where(dist < distance, dist, distance)
        m = jnp.max(distance, axis=1, keepdims=True)
        far = jnp.min(jnp.where(distance == m, lane, N), axis=1, keepdims=True)
        return distance, far.astype(jnp.int32), idx_acc, cattr_acc

    dist0 = jnp.full((B, N), 1e10, dtype=jnp.float32)
    far0 = jnp.zeros((B, 1), dtype=jnp.int32)
    idx0 = jnp.zeros((B, G), dtype=jnp.int32)
    cattr0 = jnp.zeros((A, B, G), dtype=jnp.float32)
    _, _, idx_acc, cattr_acc = jax.lax.fori_loop(
        0, G, step, (dist0, far0, idx0, cattr0))
    idx_ref[...] = idx_acc
    cattr_ref[...] = cattr_acc


def _fps(xyz_am):
    return pl.pallas_call(
        _fps_body,
        out_shape=(
            jax.ShapeDtypeStruct((B, G), jnp.int32),
            jax.ShapeDtypeStruct((A, B, G), jnp.float32),
        ),
    )(xyz_am)


NCH = 64
CW = 128
T = 6


def _topk_body(cattr_ref, xyzt_ref, idx_ref):
    a = cattr_ref[0]
    bt = xyzt_ref[0]
    aa = jnp.sum(a * a, axis=1, keepdims=True)
    bb = jnp.sum(bt * bt, axis=0, keepdims=True)
    ab = jax.lax.dot_general(a, bt, (((1,), (0,)), ((), ())),
                             preferred_element_type=jnp.float32)
    d2 = (aa + bb) - 2.0 * ab
    dist = jnp.sqrt(jnp.maximum(d2, 0.0))

    dist3 = dist.reshape(G, NCH, CW)
    gi = (jax.lax.broadcasted_iota(jnp.int32, (G, NCH, CW), 1) * CW
          + jax.lax.broadcasted_iota(jnp.int32, (G, NCH, CW), 2))
    INF = jnp.float32(jnp.inf)
    cvs, cis = [], []
    for _ in range(T):
        m3 = jnp.min(dist3, axis=2)
        sel = jnp.where(dist3 == m3[:, :, None], gi, jnp.int32(1 << 30))
        gi3 = jnp.min(sel, axis=2)
        dist3 = jnp.where(gi == gi3[:, :, None], INF, dist3)
        cvs.append(m3)
        cis.append(gi3)
    cand_v = jnp.concatenate(cvs, axis=1)
    cand_i = jnp.concatenate(cis, axis=1)

    def step(j, carry):
        cv, idx_acc = carry
        m = jnp.min(cv, axis=1, keepdims=True)
        sel2 = jnp.where(cv == m, cand_i, jnp.int32(1 << 30))
        pick = jnp.min(sel2, axis=1, keepdims=True)
        idx_acc = jnp.concatenate([idx_acc[:, 1:], pick], axis=1)
        cv = jnp.where(cand_i == pick, INF, cv)
        return cv, idx_acc

    idx0 = jnp.zeros((G, M), dtype=jnp.int32)
    _, idx_acc = jax.lax.fori_loop(0, M, step, (cand_v, idx0))
    idx_ref[0] = idx_acc


def _topk(centroids_attrs, xyz_bt):
    return pl.pallas_call(
        _topk_body,
        grid=(B,),
        in_specs=[
            pl.BlockSpec((1, G, A), lambda b: (b, 0, 0)),
            pl.BlockSpec((1, A, N), lambda b: (b, 0, 0)),
        ],
        out_specs=pl.BlockSpec((1, G, M), lambda b: (b, 0, 0)),
        out_shape=jax.ShapeDtypeStruct((B, G, M), jnp.int32),
    )(centroids_attrs, xyz_bt)


def kernel(xyz):
    xyz_am = jnp.transpose(xyz, (2, 0, 1))
    center_idx, cattr = _fps(xyz_am)
    centroids_attrs = jnp.transpose(cattr, (1, 2, 0))
    centroids_coors = centroids_attrs[:, :, :3]

    xyz_bt = jnp.transpose(xyz, (0, 2, 1))
    idx = _topk(centroids_attrs, xyz_bt)
    idx_base = jnp.arange(B)[:, None, None] * N
    flat_idx = (idx + idx_base).reshape(-1)
    neighborhood = jnp.take(xyz.reshape(B * N, A), flat_idx, axis=0)
    neighborhood = neighborhood.reshape(B, G, M, A)
    pad_zeros = neighborhood[:, :, :, 3:]
    nb = neighborhood[:, :, :, :3] - centroids_coors[:, :, None, :]
    neighborhood = jnp.concatenate((nb, pad_zeros), axis=-1)
    return (neighborhood, center_idx, centroids_attrs, centroids_coors)
